# Initial kernel scaffold; baseline (speedup 1.0000x reference)
import jax
import jax.numpy as jnp
from jax.experimental import pallas as pl


def kernel(queries, synthon_feats, synthon2rgroup, rgroup2reaction, proc_W1, proc_b1, proc_W2, proc_b2, rg0_W1, rg0_b1, rg0_W2, rg0_b2, rg1_W1, rg1_b1, rg1_W2, rg1_b2, rx0_W1, rx0_b1, rx0_W2, rx0_b2, rx1_W1, rx1_b1, rx1_W2, rx1_b2, rkey_W, rkey_b, skey_W, skey_b, rq_W1, rq_b1, rq_W2, rq_b2, sq_W1, sq_b1, sq_W2, sq_b2):
    raise NotImplementedError("write your pallas kernel here")


# speedup vs baseline: 1.8171x; 1.8171x over previous
import functools

import jax
import jax.numpy as jnp
from jax import lax
from jax.experimental import pallas as pl
from jax.experimental.pallas import tpu as pltpu

B, S, NR, R = 2048, 8192, 2048, 512
Q, H, RK, SK = 512, 1024, 128, 128
HA = H + 128


def _mm(x, w):
    return jax.lax.dot_general(x, w, (((1,), (0,)), ((), ())),
                               preferred_element_type=jnp.float32)


def _mm_t(x, w):
    return jax.lax.dot_general(x, w, (((1,), (1,)), ((), ())),
                               preferred_element_type=jnp.float32)


def _k1_body(x_ref, w1_ref, b1_ref, w2_ref, b2_ref, skw_ref, skb_ref,
             h_ref, sk_ref):
    x = x_ref[...]
    a = jax.nn.relu(_mm(x, w1_ref[...]) + b1_ref[...])
    h = _mm(a, w2_ref[...]) + b2_ref[...]
    blk = x.shape[0]
    ones_col = jnp.where(lax.broadcasted_iota(jnp.int32, (blk, 128), 1) == 0,
                         1.0, 0.0)
    h_ref[...] = jnp.concatenate([h, ones_col], axis=1)
    sk_ref[...] = _mm(x, skw_ref[...]) + skb_ref[...]


def _run_k1(synthon_feats, rg0_W1, rg0_b1, rg0_W2, rg0_b2, skey_W, skey_b):
    BLK = 512
    grid = (S // BLK,)
    full = lambda shape: pl.BlockSpec(shape, lambda i: (0,) * len(shape))
    return pl.pallas_call(
        _k1_body,
        grid=grid,
        in_specs=[
            pl.BlockSpec((BLK, Q), lambda i: (i, 0)),
            full((Q, H)), full((H,)), full((H, H)), full((H,)),
            full((Q, SK)), full((SK,)),
        ],
        out_specs=[
            pl.BlockSpec((BLK, HA), lambda i: (i, 0)),
            pl.BlockSpec((BLK, SK), lambda i: (i, 0)),
        ],
        out_shape=[
            jax.ShapeDtypeStruct((S, HA), jnp.float32),
            jax.ShapeDtypeStruct((S, SK), jnp.float32),
        ],
    )(synthon_feats, rg0_W1, rg0_b1, rg0_W2, rg0_b2, skey_W, skey_b)


def _make_segsum(n_in, n_out):
    BI = 512
    BO = 512

    def body(idx_ref, x_ref, out_ref):
        j = pl.program_id(0)
        i = pl.program_id(1)

        @pl.when(i == 0)
        def _():
            out_ref[...] = jnp.zeros_like(out_ref)

        idx = idx_ref[0, 0, :]
        local = idx - j * BO
        seg_iota = lax.broadcasted_iota(jnp.int32, (BO, BI), 0)
        oh_t = (seg_iota == local[None, :]).astype(jnp.bfloat16)
        x = x_ref[...].astype(jnp.bfloat16)
        out_ref[...] += jax.lax.dot_general(
            oh_t, x, (((1,), (0,)), ((), ())),
            preferred_element_type=jnp.float32)

    def run(data, idx):
        w = data.shape[1]
        idx3 = idx.astype(jnp.int32).reshape(n_in // BI, 1, BI)
        grid = (n_out // BO, n_in // BI)
        return pl.pallas_call(
            body,
            grid=grid,
            in_specs=[
                pl.BlockSpec((1, 1, BI), lambda j, i: (i, 0, 0)),
                pl.BlockSpec((BI, w), lambda j, i: (i, 0)),
            ],
            out_specs=pl.BlockSpec((BO, w), lambda j, i: (j, 0)),
            out_shape=jax.ShapeDtypeStruct((n_out, w), jnp.float32),
        )(idx3, data)

    return run


def _k2_body(pw_ref, w1_ref, b1_ref, w2_ref, b2_ref,
             x1_ref, c1_ref, x2_ref, c2_ref, g_ref):
    pw = pw_ref[...]
    cnt = jnp.maximum(pw[:, H], 1.0)
    pooled = pw[:, :H] / cnt[:, None]
    a = jax.nn.relu(_mm(pooled, w1_ref[...]) + b1_ref[...])
    rf = _mm(a, w2_ref[...]) + b2_ref[...]
    b = jax.nn.relu(_mm(rf, x1_ref[...]) + c1_ref[...])
    g_ref[...] = _mm(b, x2_ref[...]) + c2_ref[...]


def _run_k2(pooled_wide, rg1_W1, rg1_b1, rg1_W2, rg1_b2,
            rx0_W1, rx0_b1, rx0_W2, rx0_b2):
    BLK = 512
    grid = (NR // BLK,)
    full = lambda shape: pl.BlockSpec(shape, lambda i: (0,) * len(shape))
    return pl.pallas_call(
        _k2_body,
        grid=grid,
        in_specs=[
            pl.BlockSpec((BLK, HA), lambda i: (i, 0)),
            full((H, H)), full((H,)), full((H, Q)), full((Q,)),
            full((Q, H)), full((H,)), full((H, H)), full((H,)),
        ],
        out_specs=pl.BlockSpec((BLK, H), lambda i: (i, 0)),
        out_shape=jax.ShapeDtypeStruct((NR, H), jnp.float32),
    )(pooled_wide, rg1_W1, rg1_b1, rg1_W2, rg1_b2,
      rx0_W1, rx0_b1, rx0_W2, rx0_b2)


def _k3_body(rp_ref, w1_ref, b1_ref, w2_ref, b2_ref, kw_ref, kb_ref, out_ref):
    a = jax.nn.relu(_mm(rp_ref[...], w1_ref[...]) + b1_ref[...])
    rf = _mm(a, w2_ref[...]) + b2_ref[...]
    out_ref[...] = _mm(rf, kw_ref[...]) + kb_ref[...]


def _run_k3(reaction_pooled, rx1_W1, rx1_b1, rx1_W2, rx1_b2, rkey_W, rkey_b):
    return pl.pallas_call(
        _k3_body,
        out_shape=jax.ShapeDtypeStruct((R, RK), jnp.float32),
    )(reaction_pooled, rx1_W1, rx1_b1, rx1_W2, rx1_b2, rkey_W, rkey_b)


def _k4_body(x_ref, pw1_ref, pb1_ref, pw2_ref, pb2_ref,
             rw1_ref, rb1_ref, rw2_ref, rb2_ref,
             sw1_ref, sb1_ref, sw2_ref, sb2_ref, rq_ref, sq_ref):
    x = x_ref[...]
    a = jax.nn.relu(_mm(x, pw1_ref[...]) + pb1_ref[...])
    q = x + _mm(a, pw2_ref[...]) + pb2_ref[...]
    ar = jax.nn.relu(_mm(q, rw1_ref[...]) + rb1_ref[...])
    rq_ref[...] = _mm(ar, rw2_ref[...]) + rb2_ref[...]
    asq = jax.nn.relu(_mm(q, sw1_ref[...]) + sb1_ref[...])
    sq_ref[...] = _mm(asq, sw2_ref[...]) + sb2_ref[...]


def _run_k4(queries, proc_W1, proc_b1, proc_W2, proc_b2,
            rq_W1, rq_b1, rq_W2, rq_b2, sq_W1, sq_b1, sq_W2, sq_b2):
    BLK = 512
    grid = (B // BLK,)
    full = lambda shape: pl.BlockSpec(shape, lambda i: (0,) * len(shape))
    return pl.pallas_call(
        _k4_body,
        grid=grid,
        in_specs=[
            pl.BlockSpec((BLK, Q), lambda i: (i, 0)),
            full((Q, H)), full((H,)), full((H, Q)), full((Q,)),
            full((Q, H)), full((H,)), full((H, RK)), full((RK,)),
            full((Q, H)), full((H,)), full((H, SK)), full((SK,)),
        ],
        out_specs=[
            pl.BlockSpec((BLK, RK), lambda i: (i, 0)),
            pl.BlockSpec((BLK, SK), lambda i: (i, 0)),
        ],
        out_shape=[
            jax.ShapeDtypeStruct((B, RK), jnp.float32),
            jax.ShapeDtypeStruct((B, SK), jnp.float32),
        ],
    )(queries, proc_W1, proc_b1, proc_W2, proc_b2,
      rq_W1, rq_b1, rq_W2, rq_b2, sq_W1, sq_b1, sq_W2, sq_b2)


def _log_softmax(x):
    m = jnp.max(x, axis=1, keepdims=True)
    e = jnp.exp(x - m)
    lse = jnp.log(jnp.sum(e, axis=1, keepdims=True)) + m
    return x - lse


def _k5_body(rqa_ref, sqa_ref, rk_ref, sk_ref, out_ref):
    rl = _mm_t(rqa_ref[...], rk_ref[...])
    sl = _mm_t(sqa_ref[...], sk_ref[...])
    out_ref[:, :R] = _log_softmax(rl)
    out_ref[:, R:] = _log_softmax(sl)


def _run_k5(rq_act, sq_act, reaction_keys, synthon_keys):
    BLK = 256
    grid = (B // BLK,)
    full = lambda shape: pl.BlockSpec(shape, lambda i: (0,) * len(shape))
    return pl.pallas_call(
        _k5_body,
        grid=grid,
        in_specs=[
            pl.BlockSpec((BLK, RK), lambda i: (i, 0)),
            pl.BlockSpec((BLK, SK), lambda i: (i, 0)),
            full((R, RK)), full((S, SK)),
        ],
        out_specs=pl.BlockSpec((BLK, R + S), lambda i: (i, 0)),
        out_shape=jax.ShapeDtypeStruct((B, R + S), jnp.float32),
    )(rq_act, sq_act, reaction_keys, synthon_keys)


def kernel(queries, synthon_feats, synthon2rgroup, rgroup2reaction,
           proc_W1, proc_b1, proc_W2, proc_b2,
           rg0_W1, rg0_b1, rg0_W2, rg0_b2,
           rg1_W1, rg1_b1, rg1_W2, rg1_b2,
           rx0_W1, rx0_b1, rx0_W2, rx0_b2,
           rx1_W1, rx1_b1, rx1_W2, rx1_b2,
           rkey_W, rkey_b, skey_W, skey_b,
           rq_W1, rq_b1, rq_W2, rq_b2,
           sq_W1, sq_b1, sq_W2, sq_b2):
    h_aug, synthon_keys = _run_k1(synthon_feats, rg0_W1, rg0_b1,
                                  rg0_W2, rg0_b2, skey_W, skey_b)
    rq_act, sq_act = _run_k4(queries, proc_W1, proc_b1, proc_W2, proc_b2,
                             rq_W1, rq_b1, rq_W2, rq_b2,
                             sq_W1, sq_b1, sq_W2, sq_b2)

    pooled_wide = _make_segsum(S, NR)(h_aug, synthon2rgroup)
    g = _run_k2(pooled_wide, rg1_W1, rg1_b1, rg1_W2, rg1_b2,
                rx0_W1, rx0_b1, rx0_W2, rx0_b2)
    reaction_pooled = _make_segsum(NR, R)(g, rgroup2reaction)
    reaction_keys = _run_k3(reaction_pooled, rx1_W1, rx1_b1, rx1_W2, rx1_b2,
                            rkey_W, rkey_b)
    return _run_k5(rq_act, sq_act, reaction_keys, synthon_keys)
